# baseline (speedup 1.0000x reference)
import functools

import jax
import jax.numpy as jnp
from jax import lax
from jax.experimental import pallas as pl
from jax.experimental.pallas import tpu as pltpu
from jax.experimental.pallas import tpu_sc as plsc

_P = 0.1
_NC, _NS = 2, 16
_NW = _NC * _NS
_L = 16
_CHUNK = 512
_KSUB = _CHUNK // 128


def _build_sc_kernel(B, V, D):
    BPW = B // _NW
    NCHUNK = BPW // _CHUNK
    mesh = plsc.VectorSubcoreMesh(core_axis_name="c", subcore_axis_name="s")

    def body(w_hbm, m_hbm, idx_hbm, out_hbm, idx_v, rows_v, mval_v, gsem, msem):
        cid = lax.axis_index("c")
        sid = lax.axis_index("s")
        wid = sid * _NC + cid
        row0 = wid * (BPW // 128)

        def chunk_body(g, carry):
            ib = row0 + g * _KSUB
            pltpu.sync_copy(idx_hbm.at[pl.ds(ib, _KSUB)], idx_v)
            cps = []
            for j in range(_KSUB):
                cps.append(pltpu.async_copy(
                    w_hbm.at[idx_v.at[j]], rows_v.at[pl.ds(j * 128, 128)], gsem))
                cps.append(pltpu.async_copy(
                    m_hbm.at[idx_v.at[j]], mval_v.at[pl.ds(j * 128, 128)], msem))
            for c in cps:
                c.wait()

            @plsc.parallel_loop(0, _CHUNK, step=_L)
            def _scale(r0):
                mvec = mval_v[pl.ds(r0, _L)]
                for r in range(_L):
                    m = mvec[r]
                    for k in range(D // _L):
                        rows_v[r0 + r, pl.ds(k * _L, _L)] = (
                            rows_v[r0 + r, pl.ds(k * _L, _L)] * m)

            pltpu.sync_copy(rows_v, out_hbm.at[pl.ds(wid * BPW + g * _CHUNK, _CHUNK)])
            return carry

        lax.fori_loop(0, NCHUNK, chunk_body, 0)

    return pl.kernel(
        body,
        out_type=jax.ShapeDtypeStruct((B, D), jnp.float32),
        mesh=mesh,
        compiler_params=pltpu.CompilerParams(use_tc_tiling_on_sc=False),
        scratch_types=[
            pltpu.VMEM((_KSUB, 128), jnp.int32),
            pltpu.VMEM((_CHUNK, D), jnp.float32),
            pltpu.VMEM((_CHUNK,), jnp.float32),
            pltpu.SemaphoreType.DMA,
            pltpu.SemaphoreType.DMA,
        ],
    )


def kernel(words, weight):
    V, D = weight.shape
    B = words.size
    keep = jax.random.bernoulli(jax.random.key(42), 1.0 - _P, (V, 1))
    mask = keep.astype(weight.dtype) / (1.0 - _P)
    idx2d = words.reshape(B // 128, 128)
    out = _build_sc_kernel(B, V, D)(weight, mask.reshape(-1), idx2d)
    return out.reshape(words.shape + (D,))
